# Initial kernel scaffold; baseline (speedup 1.0000x reference)
import jax
import jax.numpy as jnp
from jax.experimental import pallas as pl


def kernel(features, edge_index, W1, b1, W2, b2, fc_W, fc_b, cl_thres):
    raise NotImplementedError("write your pallas kernel here")


# speedup vs baseline: 4.5752x; 4.5752x over previous
import functools

import jax
import jax.numpy as jnp
from jax import lax
from jax.experimental import pallas as pl
from jax.experimental.pallas import tpu as pltpu
from jax.experimental.pallas import tpu_sc as plsc

N = 10000
E = 320000
D = 128
K = 80
NSUB = 16
NCORE = 2
RPS = 632


def _row_slice(s):
    return pl.ds(jnp.minimum(s * RPS, N - RPS), RPS)

_MESH = plsc.VectorSubcoreMesh(
    core_axis_name="c", subcore_axis_name="s",
    num_cores=NCORE, num_subcores=NSUB)


@functools.partial(
    pl.kernel,
    out_type=[jax.ShapeDtypeStruct((NCORE, N, D), jnp.float32),
              jax.ShapeDtypeStruct((NCORE, N, D), jnp.float32)],
    mesh=_MESH,
    scratch_types=[
        pltpu.VMEM((2, K), jnp.int32),
        pltpu.VMEM((K, D), jnp.float32),
        pltpu.VMEM_SHARED((N, D), jnp.float32),
    ],
)
def _sc_degrees(ep_hbm, ones_hbm, zeros_hbm, osrc_hbm, odst_hbm,
                idx_v, ones_v, acc):
    c = lax.axis_index("c")
    s = lax.axis_index("s")
    wid = s * NCORE + c
    chunks_per_w = (E // K) // (NCORE * NSUB)
    base = wid * chunks_per_w
    pltpu.sync_copy(ones_hbm, ones_v)

    def _phase(row, out_hbm):
        pltpu.sync_copy(zeros_hbm.at[_row_slice(s)], acc.at[_row_slice(s)])
        plsc.subcore_barrier()

        @pl.loop(0, chunks_per_w)
        def _(i):
            pltpu.sync_copy(ep_hbm.at[base + i], idx_v)
            pltpu.sync_copy(ones_v, acc.at[idx_v.at[row]], add=True)

        plsc.subcore_barrier()
        pltpu.sync_copy(acc.at[_row_slice(s)], out_hbm.at[c, _row_slice(s)])
        plsc.subcore_barrier()

    _phase(0, osrc_hbm)
    _phase(1, odst_hbm)


@functools.partial(
    pl.kernel,
    out_type=jax.ShapeDtypeStruct((NCORE, N, D), jnp.float32),
    mesh=_MESH,
    scratch_types=[
        pltpu.VMEM((2, K), jnp.int32),
        pltpu.VMEM((K, D), jnp.float32),
        pltpu.VMEM_SHARED((N, D), jnp.float32),
        pltpu.SemaphoreType.DMA,
    ],
)
def _sc_aggregate(h_hbm, ep_hbm, zeros_hbm, out_hbm, idx_v, rows_v, acc, sem):
    c = lax.axis_index("c")
    s = lax.axis_index("s")
    wid = s * NCORE + c
    pltpu.sync_copy(zeros_hbm.at[_row_slice(s)], acc.at[_row_slice(s)])
    plsc.subcore_barrier()
    chunks_per_w = (E // K) // (NCORE * NSUB)
    base = wid * chunks_per_w

    @pl.loop(0, chunks_per_w)
    def _(i):
        pltpu.sync_copy(ep_hbm.at[base + i], idx_v)
        pltpu.async_copy(h_hbm.at[idx_v.at[0]], rows_v, sem).wait()
        pltpu.sync_copy(rows_v, acc.at[idx_v.at[1]], add=True)

    plsc.subcore_barrier()
    pltpu.sync_copy(acc.at[_row_slice(s)], out_hbm.at[c, _row_slice(s)])


def _norm(deg_ref):
    deg = deg_ref[0, :, 0:1] + deg_ref[1, :, 0:1]
    return lax.rsqrt(jnp.clip(deg, 1.0, None))


def _tc_layer1(features, W1, dsrc):
    def body(x_ref, w_ref, dsrc_ref, o_ref):
        ns = _norm(dsrc_ref)
        o_ref[...] = jnp.dot(x_ref[...] * ns, w_ref[...])

    return pl.pallas_call(
        body, out_shape=jax.ShapeDtypeStruct((N, D), jnp.float32),
    )(features, W1, dsrc)


def _tc_mid(p, dsrc, ddst, b, W):
    def body(p_ref, dsrc_ref, ddst_ref, b_ref, w_ref, o_ref):
        ns = _norm(dsrc_ref)
        nd = _norm(ddst_ref)
        h = jnp.maximum((p_ref[0] + p_ref[1]) * nd + b_ref[...], 0.0)
        o_ref[...] = jnp.dot(h * ns, w_ref[...])

    return pl.pallas_call(
        body, out_shape=jax.ShapeDtypeStruct((N, D), jnp.float32),
    )(p, dsrc, ddst, b, W)


def _tc_head(p, ddst, b, fc_W, shift):
    def body(p_ref, ddst_ref, b_ref, w_ref, shift_ref, o_ref):
        nd = _norm(ddst_ref)
        h = jnp.maximum((p_ref[0] + p_ref[1]) * nd + b_ref[...], 0.0)
        o_ref[...] = jnp.dot(h, w_ref[...]) + shift_ref[...]

    return pl.pallas_call(
        body, out_shape=jax.ShapeDtypeStruct((N, 1), jnp.float32),
    )(p, ddst, b, fc_W, shift)


def kernel(features, edge_index, W1, b1, W2, b2, fc_W, fc_b, cl_thres):
    ei = edge_index.astype(jnp.int32)
    epairs = jnp.stack(
        [ei[0].reshape(E // K, K), ei[1].reshape(E // K, K)], axis=1)
    ones_k = jnp.ones((K, D), jnp.float32)
    zeros128 = jnp.zeros((N, D), jnp.float32)

    dsrc, ddst = _sc_degrees(epairs, ones_k, zeros128)
    h1t = _tc_layer1(features, W1, dsrc)
    p1 = _sc_aggregate(h1t, epairs, zeros128)
    h2t = _tc_mid(p1, dsrc, ddst, b1, W2)
    p2 = _sc_aggregate(h2t, epairs, zeros128)
    shift = fc_b - cl_thres
    return _tc_head(p2, ddst, b2, fc_W, shift)


# speedup vs baseline: 7.5641x; 1.6533x over previous
import functools

import jax
import jax.numpy as jnp
from jax import lax
from jax.experimental import pallas as pl
from jax.experimental.pallas import tpu as pltpu
from jax.experimental.pallas import tpu_sc as plsc

N = 10000
E = 320000
D = 128
K = 80
NSUB = 16
NCORE = 2
RPS = 632


def _row_slice(s):
    return pl.ds(jnp.minimum(s * RPS, N - RPS), RPS)


CPW = (E // K) // 32
LAG = 8
_BLOCKS = (63, 62)
IB = max(_BLOCKS)

_MESH = plsc.VectorSubcoreMesh(
    core_axis_name="c", subcore_axis_name="s",
    num_cores=NCORE, num_subcores=NSUB)


@functools.partial(
    pl.kernel,
    out_type=[jax.ShapeDtypeStruct((NCORE, N, D), jnp.float32),
              jax.ShapeDtypeStruct((NCORE, N, D), jnp.float32)],
    mesh=_MESH,
    scratch_types=[
        pltpu.VMEM((CPW, 2, K), jnp.int32),
        pltpu.VMEM((K, D), jnp.float32),
        pltpu.VMEM_SHARED((N, D), jnp.float32),
        pltpu.SemaphoreType.DMA,
    ],
)
def _sc_degrees(ep_hbm, ones_hbm, zeros_hbm, osrc_hbm, odst_hbm,
                idx_v, ones_v, acc, ssem):
    c = lax.axis_index("c")
    s = lax.axis_index("s")
    wid = s * NCORE + c
    base = wid * CPW
    pltpu.sync_copy(ones_hbm, ones_v)
    pltpu.sync_copy(ep_hbm.at[pl.ds(base, CPW)], idx_v)

    def _phase(row, out_hbm):
        pltpu.sync_copy(zeros_hbm.at[_row_slice(s)], acc.at[_row_slice(s)])
        plsc.subcore_barrier()

        @pl.loop(0, CPW)
        def _(i):
            pltpu.async_copy(ones_v, acc.at[idx_v.at[i, row]], ssem,
                             add=True)

            @pl.when(i >= LAG)
            def _():
                pltpu.make_async_copy(
                    ones_v, acc.at[idx_v.at[i, row]], ssem).wait()

        @pl.loop(0, LAG)
        def _(i):
            pltpu.make_async_copy(
                ones_v, acc.at[idx_v.at[i, row]], ssem).wait()

        plsc.subcore_barrier()
        pltpu.sync_copy(acc.at[_row_slice(s)], out_hbm.at[c, _row_slice(s)])
        plsc.subcore_barrier()

    _phase(0, osrc_hbm)
    _phase(1, odst_hbm)


@functools.partial(
    pl.kernel,
    out_type=jax.ShapeDtypeStruct((NCORE, N, D), jnp.float32),
    mesh=_MESH,
    scratch_types=[
        pltpu.VMEM((IB, 2, K), jnp.int32),
        pltpu.VMEM((K, D), jnp.float32),
        pltpu.VMEM((K, D), jnp.float32),
        pltpu.VMEM_SHARED((N, D), jnp.float32),
        pltpu.SemaphoreType.DMA,
        pltpu.SemaphoreType.DMA,
        pltpu.SemaphoreType.DMA,
        pltpu.SemaphoreType.DMA,
    ],
)
def _sc_aggregate(h_hbm, ep_hbm, zeros_hbm, out_hbm,
                  idx_v, r0, r1, acc, g0, g1, s0, s1):
    c = lax.axis_index("c")
    s = lax.axis_index("s")
    wid = s * NCORE + c
    base = wid * CPW
    pltpu.sync_copy(zeros_hbm.at[_row_slice(s)], acc.at[_row_slice(s)])
    plsc.subcore_barrier()

    def _gather(i, r, sem):
        return pltpu.async_copy(h_hbm.at[idx_v.at[i, 0]], r, sem)

    def _gwait(i, r, sem):
        pltpu.make_async_copy(h_hbm.at[idx_v.at[i, 0]], r, sem).wait()

    def _scat(i, r, sem):
        return pltpu.async_copy(r, acc.at[idx_v.at[i, 1]], sem, add=True)

    def _swait(i, r, sem):
        pltpu.make_async_copy(r, acc.at[idx_v.at[i, 1]], sem).wait()

    off = 0
    for sz in _BLOCKS:
        pltpu.sync_copy(ep_hbm.at[pl.ds(base + off, sz)],
                        idx_v.at[pl.ds(0, sz)])
        _gather(0, r0, g0)
        _gather(1, r1, g1)

        @pl.loop(0, sz, step=2)
        def _(i):
            _gwait(i, r0, g0)
            _scat(i, r0, s0)

            @pl.when(i + 1 < sz)
            def _():
                _gwait(i + 1, r1, g1)
                _scat(i + 1, r1, s1)

            @pl.when(i + 2 < sz)
            def _():
                _swait(i, r0, s0)
                _gather(i + 2, r0, g0)

            @pl.when(i + 3 < sz)
            def _():
                _swait(i + 1, r1, s1)
                _gather(i + 3, r1, g1)

        if sz % 2:
            _swait(sz - 1, r0, s0)
            _swait(sz - 2, r1, s1)
        else:
            _swait(sz - 2, r0, s0)
            _swait(sz - 1, r1, s1)
        off += sz

    plsc.subcore_barrier()
    pltpu.sync_copy(acc.at[_row_slice(s)], out_hbm.at[c, _row_slice(s)])


def _norm(deg_ref):
    deg = deg_ref[0, :, 0:1] + deg_ref[1, :, 0:1]
    return lax.rsqrt(jnp.clip(deg, 1.0, None))


def _tc_layer1(features, W1, dsrc):
    def body(x_ref, w_ref, dsrc_ref, o_ref):
        ns = _norm(dsrc_ref)
        o_ref[...] = jnp.dot(x_ref[...] * ns, w_ref[...])

    return pl.pallas_call(
        body, out_shape=jax.ShapeDtypeStruct((N, D), jnp.float32),
    )(features, W1, dsrc)


def _tc_mid(p, dsrc, ddst, b, W):
    def body(p_ref, dsrc_ref, ddst_ref, b_ref, w_ref, o_ref):
        ns = _norm(dsrc_ref)
        nd = _norm(ddst_ref)
        h = jnp.maximum((p_ref[0] + p_ref[1]) * nd + b_ref[...], 0.0)
        o_ref[...] = jnp.dot(h * ns, w_ref[...])

    return pl.pallas_call(
        body, out_shape=jax.ShapeDtypeStruct((N, D), jnp.float32),
    )(p, dsrc, ddst, b, W)


def _tc_head(p, ddst, b, fc_W, shift):
    def body(p_ref, ddst_ref, b_ref, w_ref, shift_ref, o_ref):
        nd = _norm(ddst_ref)
        h = jnp.maximum((p_ref[0] + p_ref[1]) * nd + b_ref[...], 0.0)
        o_ref[...] = jnp.dot(h, w_ref[...]) + shift_ref[...]

    return pl.pallas_call(
        body, out_shape=jax.ShapeDtypeStruct((N, 1), jnp.float32),
    )(p, ddst, b, fc_W, shift)


def kernel(features, edge_index, W1, b1, W2, b2, fc_W, fc_b, cl_thres):
    ei = edge_index.astype(jnp.int32)
    epairs = jnp.stack(
        [ei[0].reshape(E // K, K), ei[1].reshape(E // K, K)], axis=1)
    ones_k = jnp.ones((K, D), jnp.float32)
    zeros128 = jnp.zeros((N, D), jnp.float32)

    dsrc, ddst = _sc_degrees(epairs, ones_k, zeros128)
    h1t = _tc_layer1(features, W1, dsrc)
    p1 = _sc_aggregate(h1t, epairs, zeros128)
    h2t = _tc_mid(p1, dsrc, ddst, b1, W2)
    p2 = _sc_aggregate(h2t, epairs, zeros128)
    shift = fc_b - cl_thres
    return _tc_head(p2, ddst, b2, fc_W, shift)


# speedup vs baseline: 8.7843x; 1.1613x over previous
import functools

import jax
import jax.numpy as jnp
from jax import lax
from jax.experimental import pallas as pl
from jax.experimental.pallas import tpu as pltpu
from jax.experimental.pallas import tpu_sc as plsc

N = 10000
E = 320000
D = 128
K = 80
NSUB = 16
NCORE = 2
RPS = 632


def _row_slice(s):
    return pl.ds(jnp.minimum(s * RPS, N - RPS), RPS)


CPW = (E // K) // 32
_BLOCKS = (63, 62)
IB = max(_BLOCKS)

_MESH = plsc.VectorSubcoreMesh(
    core_axis_name="c", subcore_axis_name="s",
    num_cores=NCORE, num_subcores=NSUB)


DEG_B = 2560


def _tc_degrees(ei):
    def body(ei_ref, osrc_ref, odst_ref):
        iota_col = lax.broadcasted_iota(jnp.int32, (128, 1), 0)

        def onehots(idx_row):
            hi = (idx_row >> 7) == iota_col
            lo = (idx_row & 127) == iota_col
            return hi.astype(jnp.bfloat16), lo.astype(jnp.bfloat16)

        dn = (((1,), (1,)), ((), ()))

        def step(i, carry):
            asrc, adst = carry
            sh, sl = onehots(ei_ref[0:1, pl.ds(i * DEG_B, DEG_B)])
            dh, dl = onehots(ei_ref[1:2, pl.ds(i * DEG_B, DEG_B)])
            asrc = asrc + lax.dot_general(
                sh, sl, dn, preferred_element_type=jnp.float32)
            adst = adst + lax.dot_general(
                dh, dl, dn, preferred_element_type=jnp.float32)
            return asrc, adst

        zero = jnp.zeros((128, 128), jnp.float32)
        asrc, adst = lax.fori_loop(0, E // DEG_B, step, (zero, zero))
        osrc_ref[...] = asrc
        odst_ref[...] = adst

    return pl.pallas_call(
        body,
        out_shape=[jax.ShapeDtypeStruct((128, 128), jnp.float32),
                   jax.ShapeDtypeStruct((128, 128), jnp.float32)],
    )(ei)


@functools.partial(
    pl.kernel,
    out_type=jax.ShapeDtypeStruct((NCORE, N, D), jnp.float32),
    mesh=_MESH,
    scratch_types=[
        pltpu.VMEM((IB, 2, K), jnp.int32),
        pltpu.VMEM((K, D), jnp.float32),
        pltpu.VMEM((K, D), jnp.float32),
        pltpu.VMEM_SHARED((N, D), jnp.float32),
        pltpu.SemaphoreType.DMA,
        pltpu.SemaphoreType.DMA,
        pltpu.SemaphoreType.DMA,
        pltpu.SemaphoreType.DMA,
    ],
)
def _sc_aggregate(h_hbm, ep_hbm, zeros_hbm, out_hbm,
                  idx_v, r0, r1, acc, g0, g1, s0, s1):
    c = lax.axis_index("c")
    s = lax.axis_index("s")
    wid = s * NCORE + c
    base = wid * CPW
    pltpu.sync_copy(zeros_hbm.at[_row_slice(s)], acc.at[_row_slice(s)])
    plsc.subcore_barrier()

    def _gather(i, r, sem):
        return pltpu.async_copy(h_hbm.at[idx_v.at[i, 0]], r, sem)

    def _gwait(i, r, sem):
        pltpu.make_async_copy(h_hbm.at[idx_v.at[i, 0]], r, sem).wait()

    def _scat(i, r, sem):
        return pltpu.async_copy(r, acc.at[idx_v.at[i, 1]], sem, add=True)

    def _swait(i, r, sem):
        pltpu.make_async_copy(r, acc.at[idx_v.at[i, 1]], sem).wait()

    off = 0
    for sz in _BLOCKS:
        pltpu.sync_copy(ep_hbm.at[pl.ds(base + off, sz)],
                        idx_v.at[pl.ds(0, sz)])
        _gather(0, r0, g0)
        _gather(1, r1, g1)

        @pl.loop(0, sz, step=2)
        def _(i):
            _gwait(i, r0, g0)
            _scat(i, r0, s0)

            @pl.when(i + 1 < sz)
            def _():
                _gwait(i + 1, r1, g1)
                _scat(i + 1, r1, s1)

            @pl.when(i + 2 < sz)
            def _():
                _swait(i, r0, s0)
                _gather(i + 2, r0, g0)

            @pl.when(i + 3 < sz)
            def _():
                _swait(i + 1, r1, s1)
                _gather(i + 3, r1, g1)

        if sz % 2:
            _swait(sz - 1, r0, s0)
            _swait(sz - 2, r1, s1)
        else:
            _swait(sz - 2, r0, s0)
            _swait(sz - 1, r1, s1)
        off += sz

    plsc.subcore_barrier()
    pltpu.sync_copy(acc.at[_row_slice(s)], out_hbm.at[c, _row_slice(s)])


def _norm(deg_ref):
    return lax.rsqrt(jnp.clip(deg_ref[...], 1.0, None))


def _tc_layer1(features, W1, dsrc):
    def body(x_ref, w_ref, dsrc_ref, o_ref):
        ns = _norm(dsrc_ref)
        o_ref[...] = jnp.dot(x_ref[...] * ns, w_ref[...])

    return pl.pallas_call(
        body, out_shape=jax.ShapeDtypeStruct((N, D), jnp.float32),
    )(features, W1, dsrc)


def _tc_mid(p, dsrc, ddst, b, W):
    def body(p_ref, dsrc_ref, ddst_ref, b_ref, w_ref, o_ref):
        ns = _norm(dsrc_ref)
        nd = _norm(ddst_ref)
        h = jnp.maximum((p_ref[0] + p_ref[1]) * nd + b_ref[...], 0.0)
        o_ref[...] = jnp.dot(h * ns, w_ref[...])

    return pl.pallas_call(
        body, out_shape=jax.ShapeDtypeStruct((N, D), jnp.float32),
    )(p, dsrc, ddst, b, W)


def _tc_head(p, ddst, b, fc_W, shift):
    def body(p_ref, ddst_ref, b_ref, w_ref, shift_ref, o_ref):
        nd = _norm(ddst_ref)
        h = jnp.maximum((p_ref[0] + p_ref[1]) * nd + b_ref[...], 0.0)
        o_ref[...] = jnp.dot(h, w_ref[...]) + shift_ref[...]

    return pl.pallas_call(
        body, out_shape=jax.ShapeDtypeStruct((N, 1), jnp.float32),
    )(p, ddst, b, fc_W, shift)


def kernel(features, edge_index, W1, b1, W2, b2, fc_W, fc_b, cl_thres):
    ei = edge_index.astype(jnp.int32)
    epairs = jnp.stack(
        [ei[0].reshape(E // K, K), ei[1].reshape(E // K, K)], axis=1)
    zeros128 = jnp.zeros((N, D), jnp.float32)

    s2d, d2d = _tc_degrees(ei)
    dsrc = s2d.reshape(-1)[:N, None]
    ddst = d2d.reshape(-1)[:N, None]
    h1t = _tc_layer1(features, W1, dsrc)
    p1 = _sc_aggregate(h1t, epairs, zeros128)
    h2t = _tc_mid(p1, dsrc, ddst, b1, W2)
    p2 = _sc_aggregate(h2t, epairs, zeros128)
    shift = fc_b - cl_thres
    return _tc_head(p2, ddst, b2, fc_W, shift)


# speedup vs baseline: 10.1218x; 1.1523x over previous
import functools

import jax
import jax.numpy as jnp
from jax import lax
from jax.experimental import pallas as pl
from jax.experimental.pallas import tpu as pltpu
from jax.experimental.pallas import tpu_sc as plsc

N = 10000
E = 320000
D = 128
K = 80
NSUB = 16
NCORE = 2
RPS = 632


def _row_slice(s):
    return pl.ds(jnp.minimum(s * RPS, N - RPS), RPS)


CPW = (E // K) // 32
_BLOCKS = (63, 62)
IB = max(_BLOCKS)

_MESH = plsc.VectorSubcoreMesh(
    core_axis_name="c", subcore_axis_name="s",
    num_cores=NCORE, num_subcores=NSUB)


DEG_B = 2560


def _tc_degrees(ei):
    def body(ei_ref, osrc_ref, odst_ref):
        iota_col = lax.broadcasted_iota(jnp.int32, (128, 1), 0)

        def onehots(idx_row):
            hi = (idx_row >> 7) == iota_col
            lo = (idx_row & 127) == iota_col
            return hi.astype(jnp.bfloat16), lo.astype(jnp.bfloat16)

        dn = (((1,), (1,)), ((), ()))

        def step(i, carry):
            asrc, adst = carry
            sh, sl = onehots(ei_ref[0:1, pl.ds(i * DEG_B, DEG_B)])
            dh, dl = onehots(ei_ref[1:2, pl.ds(i * DEG_B, DEG_B)])
            asrc = asrc + lax.dot_general(
                sh, sl, dn, preferred_element_type=jnp.float32)
            adst = adst + lax.dot_general(
                dh, dl, dn, preferred_element_type=jnp.float32)
            return asrc, adst

        zero = jnp.zeros((128, 128), jnp.float32)
        asrc, adst = lax.fori_loop(0, E // DEG_B, step, (zero, zero))
        osrc_ref[...] = asrc
        odst_ref[...] = adst

    return pl.pallas_call(
        body,
        out_shape=[jax.ShapeDtypeStruct((128, 128), jnp.float32),
                   jax.ShapeDtypeStruct((128, 128), jnp.float32)],
    )(ei)


@functools.partial(
    pl.kernel,
    out_type=jax.ShapeDtypeStruct((NCORE, N, D), jnp.float32),
    mesh=_MESH,
    scratch_types=[
        pltpu.VMEM((IB, 2, K), jnp.int32),
        pltpu.VMEM((K, D), jnp.float32),
        pltpu.VMEM((K, D), jnp.float32),
        pltpu.VMEM((K, D), jnp.float32),
        pltpu.VMEM_SHARED((N, D), jnp.float32),
        pltpu.SemaphoreType.DMA,
        pltpu.SemaphoreType.DMA,
        pltpu.SemaphoreType.DMA,
        pltpu.SemaphoreType.DMA,
        pltpu.SemaphoreType.DMA,
        pltpu.SemaphoreType.DMA,
    ],
)
def _sc_aggregate(h_hbm, ep_hbm, zeros_hbm, out_hbm,
                  idx_v, r0, r1, r2, acc, g0, g1, g2, s0, s1, s2):
    c = lax.axis_index("c")
    s = lax.axis_index("s")
    wid = s * NCORE + c
    base = wid * CPW
    pltpu.sync_copy(zeros_hbm.at[_row_slice(s)], acc.at[_row_slice(s)])
    plsc.subcore_barrier()

    rs = (r0, r1, r2)
    gs = (g0, g1, g2)
    ss = (s0, s1, s2)

    def _gather(i, b):
        return pltpu.async_copy(h_hbm.at[idx_v.at[i, 0]], rs[b], gs[b])

    def _gwait(i, b):
        pltpu.make_async_copy(h_hbm.at[idx_v.at[i, 0]], rs[b], gs[b]).wait()

    def _scat(i, b):
        return pltpu.async_copy(rs[b], acc.at[idx_v.at[i, 1]], ss[b],
                                add=True)

    def _swait(i, b):
        pltpu.make_async_copy(rs[b], acc.at[idx_v.at[i, 1]], ss[b]).wait()

    off = 0
    for sz in _BLOCKS:
        pltpu.sync_copy(ep_hbm.at[pl.ds(base + off, sz)],
                        idx_v.at[pl.ds(0, sz)])
        for b in range(3):
            _gather(b, b)

        @pl.loop(0, sz, step=3)
        def _(i):
            for b in range(3):
                @pl.when(i + b < sz)
                def _(b=b):
                    _gwait(i + b, b)
                    _scat(i + b, b)
            for b in range(3):
                @pl.when(i + b + 3 < sz)
                def _(b=b):
                    _swait(i + b, b)
                    _gather(i + b + 3, b)

        for j in (sz - 3, sz - 2, sz - 1):
            _swait(j, j % 3)
        off += sz

    plsc.subcore_barrier()
    pltpu.sync_copy(acc.at[_row_slice(s)], out_hbm.at[c, _row_slice(s)])


def _norm(deg_ref):
    return lax.rsqrt(jnp.clip(deg_ref[...], 1.0, None))


def _tc_layer1(features, W1, dsrc):
    def body(x_ref, w_ref, dsrc_ref, o_ref):
        ns = _norm(dsrc_ref)
        o_ref[...] = jnp.dot(x_ref[...] * ns, w_ref[...])

    return pl.pallas_call(
        body, out_shape=jax.ShapeDtypeStruct((N, D), jnp.float32),
    )(features, W1, dsrc)


def _tc_mid(p, dsrc, ddst, b, W):
    def body(p_ref, dsrc_ref, ddst_ref, b_ref, w_ref, o_ref):
        ns = _norm(dsrc_ref)
        nd = _norm(ddst_ref)
        h = jnp.maximum((p_ref[0] + p_ref[1]) * nd + b_ref[...], 0.0)
        o_ref[...] = jnp.dot(h * ns, w_ref[...])

    return pl.pallas_call(
        body, out_shape=jax.ShapeDtypeStruct((N, D), jnp.float32),
    )(p, dsrc, ddst, b, W)


def _tc_head(p, ddst, b, fc_W, shift):
    def body(p_ref, ddst_ref, b_ref, w_ref, shift_ref, o_ref):
        nd = _norm(ddst_ref)
        h = jnp.maximum((p_ref[0] + p_ref[1]) * nd + b_ref[...], 0.0)
        o_ref[...] = jnp.dot(h, w_ref[...]) + shift_ref[...]

    return pl.pallas_call(
        body, out_shape=jax.ShapeDtypeStruct((N, 1), jnp.float32),
    )(p, ddst, b, fc_W, shift)


def kernel(features, edge_index, W1, b1, W2, b2, fc_W, fc_b, cl_thres):
    ei = edge_index.astype(jnp.int32)
    epairs = jnp.stack(
        [ei[0].reshape(E // K, K), ei[1].reshape(E // K, K)], axis=1)
    zeros128 = jnp.zeros((N, D), jnp.float32)

    s2d, d2d = _tc_degrees(ei)
    dsrc = s2d.reshape(-1)[:N, None]
    ddst = d2d.reshape(-1)[:N, None]
    h1t = _tc_layer1(features, W1, dsrc)
    p1 = _sc_aggregate(h1t, epairs, zeros128)
    h2t = _tc_mid(p1, dsrc, ddst, b1, W2)
    p2 = _sc_aggregate(h2t, epairs, zeros128)
    shift = fc_b - cl_thres
    return _tc_head(p2, ddst, b2, fc_W, shift)
